# baseline (speedup 1.0000x reference)
import jax
import jax.numpy as jnp
from jax.experimental import pallas as pl
from jax.experimental.pallas import tpu as pltpu

_ALPHA = 0.2


def _pick_block(n: int, cap: int) -> int:
    best = 0
    for b in range(8, cap + 1, 8):
        if n % b == 0:
            best = b
    return best if best else n


def _feat_kernel(x_ref, w_ref, a_ref, h_ref, exps_ref):
    d = w_ref.shape[1]
    h = jnp.dot(x_ref[...], w_ref[...], preferred_element_type=jnp.float32)
    h_ref[...] = h
    f1 = jnp.sum(h * a_ref[0:1, :d], axis=1, keepdims=True)
    f2 = jnp.sum(h * a_ref[0:1, d:], axis=1, keepdims=True)
    exps_ref[...] = jnp.concatenate(
        [jnp.exp(-f1), jnp.exp(-_ALPHA * f1),
         jnp.exp(-f2), jnp.exp(-_ALPHA * f2)], axis=1)


def _gat_kernel(adj_ref, h_hbm, u_ref, vt_hbm, out_ref, h_vmem, vt_vmem, sem):
    @pl.when(pl.program_id(0) == 0)
    def _load_resident():
        cp_h = pltpu.make_async_copy(h_hbm, h_vmem, sem)
        cp_h.start()
        cp_h.wait()
        cp_v = pltpu.make_async_copy(vt_hbm, vt_vmem, sem)
        cp_v.start()
        cp_v.wait()

    u1 = u_ref[:, 0:1]
    u2 = u_ref[:, 1:2]
    v1 = vt_vmem[2:3, :]
    v2 = vt_vmem[3:4, :]
    e = adj_ref[...] * jnp.minimum(u1 * v1, u2 * v2)
    rs = jnp.sum(e, axis=1, keepdims=True)
    acc = jnp.dot(e, h_vmem[...], preferred_element_type=jnp.float32)
    y = acc / rs
    out_ref[...] = jnp.where(y > 0, y, jnp.exp(y) - 1.0)


def kernel(input, adj, W, a):
    n, d_in = input.shape
    d_out = W.shape[1]

    brh = _pick_block(n, 1024)
    h, exps = pl.pallas_call(
        _feat_kernel,
        grid=(n // brh,),
        in_specs=[
            pl.BlockSpec((brh, d_in), lambda i: (i, 0)),
            pl.BlockSpec((d_in, d_out), lambda i: (0, 0)),
            pl.BlockSpec((1, 2 * d_out), lambda i: (0, 0)),
        ],
        out_specs=[
            pl.BlockSpec((brh, d_out), lambda i: (i, 0)),
            pl.BlockSpec((brh, 4), lambda i: (i, 0)),
        ],
        out_shape=[
            jax.ShapeDtypeStruct((n, d_out), jnp.float32),
            jax.ShapeDtypeStruct((n, 4), jnp.float32),
        ],
    )(input, W, a)

    exps_t = exps.T

    br = _pick_block(n, 200)
    out = pl.pallas_call(
        _gat_kernel,
        grid=(n // br,),
        in_specs=[
            pl.BlockSpec((br, n), lambda i: (i, 0)),
            pl.BlockSpec(memory_space=pltpu.MemorySpace.HBM),
            pl.BlockSpec((br, 4), lambda i: (i, 0)),
            pl.BlockSpec(memory_space=pltpu.MemorySpace.HBM),
        ],
        out_specs=pl.BlockSpec((br, d_out), lambda i: (i, 0)),
        out_shape=jax.ShapeDtypeStruct((n, d_out), jnp.float32),
        scratch_shapes=[
            pltpu.VMEM((n, d_out), jnp.float32),
            pltpu.VMEM((4, n), jnp.float32),
            pltpu.SemaphoreType.DMA,
        ],
        compiler_params=pltpu.CompilerParams(
            vmem_limit_bytes=100 * 1024 * 1024),
    )(adj, h, exps, exps_t)
    return out


# speedup vs baseline: 1.2172x; 1.2172x over previous
import jax
import jax.numpy as jnp
from jax.experimental import pallas as pl
from jax.experimental.pallas import tpu as pltpu

_ALPHA = 0.2


def _pick_block(n: int, cap: int) -> int:
    best = 0
    for b in range(8, cap + 1, 8):
        if n % b == 0:
            best = b
    return best if best else n


def _feat_kernel(x_ref, w_ref, a_ref, hext_ref, exps_ref):
    d = w_ref.shape[1]
    h = jnp.dot(x_ref[...], w_ref[...], preferred_element_type=jnp.float32)
    hext_ref[...] = jnp.concatenate(
        [h, jnp.ones_like(h[:, :1]),
         jnp.zeros_like(h[:, : d - 1])], axis=1).astype(jnp.bfloat16)
    f1 = jnp.sum(h * a_ref[0:1, :d], axis=1, keepdims=True)
    f2 = jnp.sum(h * a_ref[0:1, d:], axis=1, keepdims=True)
    exps_ref[...] = jnp.concatenate(
        [jnp.exp(-f1), jnp.exp(-_ALPHA * f1),
         jnp.exp(-f2), jnp.exp(-_ALPHA * f2)], axis=1)


def _gat_kernel(adj_ref, h_hbm, u_ref, vt_hbm, out_ref, h_vmem, vt_vmem, sem):
    @pl.when(pl.program_id(0) == 0)
    def _load_resident():
        cp_h = pltpu.make_async_copy(h_hbm, h_vmem, sem)
        cp_h.start()
        cp_h.wait()
        cp_v = pltpu.make_async_copy(vt_hbm, vt_vmem, sem)
        cp_v.start()
        cp_v.wait()

    u1 = u_ref[:, 0:1]
    u2 = u_ref[:, 1:2]
    v1 = vt_vmem[2:3, :]
    v2 = vt_vmem[3:4, :]
    d = out_ref.shape[1]
    e = (adj_ref[...] * jnp.minimum(u1 * v1, u2 * v2)).astype(jnp.bfloat16)
    acc = jnp.dot(e, h_vmem[...], preferred_element_type=jnp.float32)
    y = acc[:, :d] / acc[:, d:d + 1]
    out_ref[...] = jnp.where(y > 0, y, jnp.exp(y) - 1.0)


def kernel(input, adj, W, a):
    n, d_in = input.shape
    d_out = W.shape[1]

    brh = _pick_block(n, 1024)
    hext, exps = pl.pallas_call(
        _feat_kernel,
        grid=(n // brh,),
        in_specs=[
            pl.BlockSpec((brh, d_in), lambda i: (i, 0)),
            pl.BlockSpec((d_in, d_out), lambda i: (0, 0)),
            pl.BlockSpec((1, 2 * d_out), lambda i: (0, 0)),
        ],
        out_specs=[
            pl.BlockSpec((brh, 2 * d_out), lambda i: (i, 0)),
            pl.BlockSpec((brh, 4), lambda i: (i, 0)),
        ],
        out_shape=[
            jax.ShapeDtypeStruct((n, 2 * d_out), jnp.bfloat16),
            jax.ShapeDtypeStruct((n, 4), jnp.float32),
        ],
    )(input, W, a)

    exps_t = exps.T

    br = _pick_block(n, 400)
    out = pl.pallas_call(
        _gat_kernel,
        grid=(n // br,),
        in_specs=[
            pl.BlockSpec((br, n), lambda i: (i, 0)),
            pl.BlockSpec(memory_space=pltpu.MemorySpace.HBM),
            pl.BlockSpec((br, 4), lambda i: (i, 0)),
            pl.BlockSpec(memory_space=pltpu.MemorySpace.HBM),
        ],
        out_specs=pl.BlockSpec((br, d_out), lambda i: (i, 0)),
        out_shape=jax.ShapeDtypeStruct((n, d_out), jnp.float32),
        scratch_shapes=[
            pltpu.VMEM((n, 2 * d_out), jnp.bfloat16),
            pltpu.VMEM((4, n), jnp.float32),
            pltpu.SemaphoreType.DMA,
        ],
        compiler_params=pltpu.CompilerParams(
            vmem_limit_bytes=100 * 1024 * 1024),
    )(adj, hext, exps, exps_t)
    return out


# speedup vs baseline: 1.2291x; 1.0098x over previous
import jax
import jax.numpy as jnp
from jax.experimental import pallas as pl
from jax.experimental.pallas import tpu as pltpu

_ALPHA = 0.2


def _pick_block(n: int, cap: int) -> int:
    best = 0
    for b in range(8, cap + 1, 8):
        if n % b == 0:
            best = b
    return best if best else n


def _feat_kernel(x_ref, w_ref, a_ref, hext_ref, exps_ref):
    d = w_ref.shape[1]
    h = jnp.dot(x_ref[...], w_ref[...], preferred_element_type=jnp.float32)
    hext_ref[...] = jnp.concatenate(
        [h, jnp.ones_like(h[:, :1]),
         jnp.zeros_like(h[:, : d - 1])], axis=1).astype(jnp.bfloat16)
    f1 = jnp.sum(h * a_ref[0:1, :d], axis=1, keepdims=True)
    f2 = jnp.sum(h * a_ref[0:1, d:], axis=1, keepdims=True)
    exps_ref[...] = jnp.concatenate(
        [jnp.exp(-f1), jnp.exp(-_ALPHA * f1),
         jnp.exp(-f2), jnp.exp(-_ALPHA * f2)], axis=1)


def _gat_kernel(adj_ref, h_hbm, u_ref, vt_hbm, out_ref, h_vmem, vt_vmem, sem,
                vsem):
    @pl.when(pl.program_id(0) == 0)
    def _load_resident():
        cp_h = pltpu.make_async_copy(h_hbm, h_vmem, sem)
        cp_h.start()
        cp_v = pltpu.make_async_copy(vt_hbm, vt_vmem, vsem)
        cp_v.start()
        cp_h.wait()
        cp_v.wait()

    u1 = u_ref[:, 0:1]
    u2 = u_ref[:, 1:2]
    v1 = vt_vmem[2:3, :]
    v2 = vt_vmem[3:4, :]
    d = out_ref.shape[1]
    e = (adj_ref[...] * jnp.minimum(u1 * v1, u2 * v2)).astype(jnp.bfloat16)
    acc = jnp.dot(e, h_vmem[...], preferred_element_type=jnp.float32)
    y = acc[:, :d] / acc[:, d:d + 1]
    out_ref[...] = jnp.where(y > 0, y, jnp.exp(y) - 1.0)


def kernel(input, adj, W, a):
    n, d_in = input.shape
    d_out = W.shape[1]

    brh = _pick_block(n, 1024)
    hext, exps = pl.pallas_call(
        _feat_kernel,
        grid=(n // brh,),
        in_specs=[
            pl.BlockSpec((brh, d_in), lambda i: (i, 0)),
            pl.BlockSpec((d_in, d_out), lambda i: (0, 0)),
            pl.BlockSpec((1, 2 * d_out), lambda i: (0, 0)),
        ],
        out_specs=[
            pl.BlockSpec((brh, 2 * d_out), lambda i: (i, 0)),
            pl.BlockSpec((brh, 4), lambda i: (i, 0)),
        ],
        out_shape=[
            jax.ShapeDtypeStruct((n, 2 * d_out), jnp.bfloat16),
            jax.ShapeDtypeStruct((n, 4), jnp.float32),
        ],
    )(input, W, a)

    exps_t = exps.T

    br = _pick_block(n, 400)
    out = pl.pallas_call(
        _gat_kernel,
        grid=(n // br,),
        in_specs=[
            pl.BlockSpec((br, n), lambda i: (i, 0)),
            pl.BlockSpec(memory_space=pltpu.MemorySpace.HBM),
            pl.BlockSpec((br, 4), lambda i: (i, 0)),
            pl.BlockSpec(memory_space=pltpu.MemorySpace.HBM),
        ],
        out_specs=pl.BlockSpec((br, d_out), lambda i: (i, 0)),
        out_shape=jax.ShapeDtypeStruct((n, d_out), jnp.float32),
        scratch_shapes=[
            pltpu.VMEM((n, 2 * d_out), jnp.bfloat16),
            pltpu.VMEM((4, n), jnp.float32),
            pltpu.SemaphoreType.DMA,
            pltpu.SemaphoreType.DMA,
        ],
        compiler_params=pltpu.CompilerParams(
            vmem_limit_bytes=100 * 1024 * 1024),
    )(adj, hext, exps, exps_t)
    return out


# speedup vs baseline: 1.2821x; 1.0431x over previous
import jax
import jax.numpy as jnp
from jax.experimental import pallas as pl
from jax.experimental.pallas import tpu as pltpu

_ALPHA = 0.2


def _pick_block(n: int, cap: int) -> int:
    best = 0
    for b in range(8, cap + 1, 8):
        if n % b == 0:
            best = b
    return best if best else n


def _feat_kernel(x_ref, w_ref, a_ref, hext_ref, exps_ref, expst_ref):
    d = w_ref.shape[1]
    h = jnp.dot(x_ref[...], w_ref[...], preferred_element_type=jnp.float32)
    hext_ref[...] = jnp.concatenate(
        [h, jnp.ones_like(h[:, :1]),
         jnp.zeros_like(h[:, : d - 1])], axis=1).astype(jnp.bfloat16)
    f1 = jnp.sum(h * a_ref[0:1, :d], axis=1, keepdims=True)
    f2 = jnp.sum(h * a_ref[0:1, d:], axis=1, keepdims=True)
    exps = jnp.concatenate(
        [jnp.exp(-f1), jnp.exp(-_ALPHA * f1),
         jnp.exp(-f2), jnp.exp(-_ALPHA * f2)], axis=1)
    exps_ref[...] = exps
    expst_ref[...] = exps.T


def _gat_kernel(adj_ref, h_hbm, u_ref, vt_hbm, out_ref, h_vmem, vt_vmem, sem,
                vsem):
    @pl.when(pl.program_id(0) == 0)
    def _load_resident():
        cp_h = pltpu.make_async_copy(h_hbm, h_vmem, sem)
        cp_h.start()
        cp_v = pltpu.make_async_copy(vt_hbm, vt_vmem, vsem)
        cp_v.start()
        cp_h.wait()
        cp_v.wait()

    u1 = u_ref[:, 0:1]
    u2 = u_ref[:, 1:2]
    v1 = vt_vmem[2:3, :]
    v2 = vt_vmem[3:4, :]
    d = out_ref.shape[1]
    e = (adj_ref[...] * jnp.minimum(u1 * v1, u2 * v2)).astype(jnp.bfloat16)
    acc = jnp.dot(e, h_vmem[...], preferred_element_type=jnp.float32)
    y = acc[:, :d] / acc[:, d:d + 1]
    out_ref[...] = jnp.where(y > 0, y, jnp.exp(y) - 1.0)


def kernel(input, adj, W, a):
    n, d_in = input.shape
    d_out = W.shape[1]

    brh = n
    hext, exps, exps_t = pl.pallas_call(
        _feat_kernel,
        grid=(n // brh,),
        in_specs=[
            pl.BlockSpec((brh, d_in), lambda i: (i, 0)),
            pl.BlockSpec((d_in, d_out), lambda i: (0, 0)),
            pl.BlockSpec((1, 2 * d_out), lambda i: (0, 0)),
        ],
        out_specs=[
            pl.BlockSpec((brh, 2 * d_out), lambda i: (i, 0)),
            pl.BlockSpec((brh, 4), lambda i: (i, 0)),
            pl.BlockSpec((4, brh), lambda i: (0, i)),
        ],
        out_shape=[
            jax.ShapeDtypeStruct((n, 2 * d_out), jnp.bfloat16),
            jax.ShapeDtypeStruct((n, 4), jnp.float32),
            jax.ShapeDtypeStruct((4, n), jnp.float32),
        ],
    )(input, W, a)

    br = _pick_block(n, 400)
    out = pl.pallas_call(
        _gat_kernel,
        grid=(n // br,),
        in_specs=[
            pl.BlockSpec((br, n), lambda i: (i, 0)),
            pl.BlockSpec(memory_space=pltpu.MemorySpace.HBM),
            pl.BlockSpec((br, 4), lambda i: (i, 0)),
            pl.BlockSpec(memory_space=pltpu.MemorySpace.HBM),
        ],
        out_specs=pl.BlockSpec((br, d_out), lambda i: (i, 0)),
        out_shape=jax.ShapeDtypeStruct((n, d_out), jnp.float32),
        scratch_shapes=[
            pltpu.VMEM((n, 2 * d_out), jnp.bfloat16),
            pltpu.VMEM((4, n), jnp.float32),
            pltpu.SemaphoreType.DMA,
            pltpu.SemaphoreType.DMA,
        ],
        compiler_params=pltpu.CompilerParams(
            vmem_limit_bytes=100 * 1024 * 1024),
    )(adj, hext, exps, exps_t)
    return out


# speedup vs baseline: 1.3513x; 1.0540x over previous
import jax
import jax.numpy as jnp
from jax.experimental import pallas as pl
from jax.experimental.pallas import tpu as pltpu

_ALPHA = 0.2


def _pick_block(n: int, cap: int) -> int:
    best = 0
    for b in range(8, cap + 1, 8):
        if n % b == 0:
            best = b
    return best if best else n


def _gat_kernel(adj_ref, x_hbm, w_ref, a_ref, out_ref,
                x_vmem, hext_vmem, exps_vmem, vt_vmem, sem):
    i = pl.program_id(0)
    br = out_ref.shape[0]
    d = out_ref.shape[1]

    @pl.when(i == 0)
    def _prologue():
        cp_x = pltpu.make_async_copy(x_hbm, x_vmem, sem)
        cp_x.start()
        cp_x.wait()
        n = x_vmem.shape[0]
        chunk = _pick_block(n, 2048)

        def body(k, _):
            xs = x_vmem[pl.ds(k * chunk, chunk), :]
            h = jnp.dot(xs, w_ref[...], preferred_element_type=jnp.float32)
            hext_vmem[pl.ds(k * chunk, chunk), :] = jnp.concatenate(
                [h, jnp.ones_like(h[:, :1]),
                 jnp.zeros_like(h[:, : d - 1])], axis=1).astype(jnp.bfloat16)
            f1 = jnp.sum(h * a_ref[0:1, :d], axis=1, keepdims=True)
            f2 = jnp.sum(h * a_ref[0:1, d:], axis=1, keepdims=True)
            exps = jnp.concatenate(
                [jnp.exp(-f1), jnp.exp(-_ALPHA * f1),
                 jnp.exp(-f2), jnp.exp(-_ALPHA * f2)], axis=1)
            exps_vmem[pl.ds(k * chunk, chunk), :] = exps
            return 0

        jax.lax.fori_loop(0, n // chunk, body, 0)
        vt_vmem[...] = exps_vmem[...].T

    @pl.when(i > 0)
    def _block():
        u = exps_vmem[pl.ds((i - 1) * br, br), :]
        u1 = u[:, 0:1]
        u2 = u[:, 1:2]
        v1 = vt_vmem[2:3, :]
        v2 = vt_vmem[3:4, :]
        e = (adj_ref[...] * jnp.minimum(u1 * v1, u2 * v2)).astype(jnp.bfloat16)
        acc = jnp.dot(e, hext_vmem[...], preferred_element_type=jnp.float32)
        y = acc[:, :d] / acc[:, d:d + 1]
        out_ref[...] = jnp.where(y > 0, y, jnp.exp(y) - 1.0)


def kernel(input, adj, W, a):
    n, d_in = input.shape
    d_out = W.shape[1]
    br = _pick_block(n, 400)
    nb = n // br

    def _blk(i):
        j = jnp.maximum(i - 1, 0)
        return (j, 0)

    out = pl.pallas_call(
        _gat_kernel,
        grid=(nb + 1,),
        in_specs=[
            pl.BlockSpec((br, n), _blk),
            pl.BlockSpec(memory_space=pltpu.MemorySpace.HBM),
            pl.BlockSpec((d_in, d_out), lambda i: (0, 0)),
            pl.BlockSpec((1, 2 * d_out), lambda i: (0, 0)),
        ],
        out_specs=pl.BlockSpec((br, d_out), _blk),
        out_shape=jax.ShapeDtypeStruct((n, d_out), jnp.float32),
        scratch_shapes=[
            pltpu.VMEM((n, d_in), jnp.float32),
            pltpu.VMEM((n, 2 * d_out), jnp.bfloat16),
            pltpu.VMEM((n, 4), jnp.float32),
            pltpu.VMEM((4, n), jnp.float32),
            pltpu.SemaphoreType.DMA,
        ],
        compiler_params=pltpu.CompilerParams(
            vmem_limit_bytes=100 * 1024 * 1024),
    )(adj, input, W, a)
    return out
